# baseline (speedup 1.0000x reference)
import functools

import jax
import jax.numpy as jnp
from jax import lax
from jax.experimental import pallas as pl
from jax.experimental.pallas import tpu as pltpu
from jax.experimental.pallas import tpu_sc as plsc

UPDATE_N = 100000
VOCAB_N = 1000000
D = 32
L = 16
NC, NS = 2, 16
NW = NC * NS
B_ROWS = 4096 * 200
ROWS_PER_W = B_ROWS // NW
CHUNK = 512
N_CHUNKS = ROWS_PER_W // CHUNK
G = 128
NG = CHUNK // G
DUMP = B_ROWS

_mesh = plsc.VectorSubcoreMesh(core_axis_name="c", subcore_axis_name="s")


@functools.partial(
    pl.kernel,
    out_type=jax.ShapeDtypeStruct((B_ROWS + 8, D), jnp.float32),
    mesh=_mesh,
    compiler_params=pltpu.CompilerParams(use_tc_tiling_on_sc=False),
    scratch_types=[
        pltpu.VMEM((ROWS_PER_W,), jnp.int32),
        pltpu.VMEM((ROWS_PER_W,), jnp.int32),
        pltpu.VMEM((ROWS_PER_W // G, G), jnp.int32),
        pltpu.VMEM((CHUNK, D), jnp.float32),
        pltpu.VMEM((CHUNK, D), jnp.float32),
        pltpu.SemaphoreType.DMA,
        pltpu.SemaphoreType.DMA,
    ],
)
def _sc_lookup(idx_hbm, wu_hbm, wf_hbm, out_hbm,
               idxv, uidxv, dstv, ubuf, fbuf, sem_g, sem_s):
    wid = lax.axis_index("s") * NC + lax.axis_index("c")
    base = wid * ROWS_PER_W

    pltpu.sync_copy(idx_hbm.at[pl.ds(base, ROWS_PER_W)], idxv)

    lanes = lax.iota(jnp.int32, L)

    def remap_body(jr, carry):
        for jc in range(G // L):
            sl = pl.ds(jr * G + jc * L, L)
            v = idxv[sl]
            uidxv[sl] = jnp.minimum(v, UPDATE_N - 1)
            self_pos = base + jr * G + jc * L + lanes
            dstv[jr, pl.ds(jc * L, L)] = jnp.where(
                v < UPDATE_N, DUMP, self_pos)
        return carry

    lax.fori_loop(0, ROWS_PER_W // G, remap_body, 0)

    def chunk_body(ci, carry):
        start = ci * CHUNK
        copies = []
        for g in range(NG):
            sl = pl.ds(start + g * G, G)
            dl = pl.ds(g * G, G)
            copies.append(pltpu.async_copy(
                wu_hbm.at[uidxv.at[sl]], ubuf.at[dl], sem_g))
            copies.append(pltpu.async_copy(
                wf_hbm.at[idxv.at[sl]], fbuf.at[dl], sem_g))
        for cp in copies:
            cp.wait()

        pltpu.sync_copy(ubuf, out_hbm.at[pl.ds(base + start, CHUNK)])

        scatters = []
        for g in range(NG):
            row = ci * NG + g
            dl = pl.ds(g * G, G)
            scatters.append(pltpu.async_copy(
                fbuf.at[dl], out_hbm.at[dstv.at[row]], sem_s))
        for sc in scatters:
            sc.wait()
        return carry

    lax.fori_loop(0, N_CHUNKS, chunk_body, 0)


def kernel(inp, W_update, W_fixed):
    idx = inp.reshape(B_ROWS).astype(jnp.int32)
    out = _sc_lookup(idx, W_update, W_fixed)
    return out[:B_ROWS].reshape(inp.shape[0], inp.shape[1], D)


# speedup vs baseline: 7.5787x; 7.5787x over previous
import functools

import jax
import jax.numpy as jnp
from jax import lax
from jax.experimental import pallas as pl
from jax.experimental.pallas import tpu as pltpu
from jax.experimental.pallas import tpu_sc as plsc

UPDATE_N = 100000
VOCAB_N = 1000000
D = 32
L = 16
NC, NS = 2, 16
NW = NC * NS
B_ROWS = 4096 * 200
ROWS_PER_W = B_ROWS // NW
CHUNK = 1024
N_CHUNKS = ROWS_PER_W // CHUNK
SUB = 8

_mesh = plsc.VectorSubcoreMesh(core_axis_name="c", subcore_axis_name="s")


@functools.partial(
    pl.kernel,
    out_type=jax.ShapeDtypeStruct((B_ROWS, D), jnp.float32),
    mesh=_mesh,
    compiler_params=pltpu.CompilerParams(use_tc_tiling_on_sc=False),
    scratch_types=[
        pltpu.VMEM((ROWS_PER_W,), jnp.int32),
        pltpu.VMEM((CHUNK, D), jnp.float32),
        pltpu.VMEM((CHUNK, D), jnp.float32),
        pltpu.SemaphoreType.DMA,
        pltpu.SemaphoreType.DMA,
        pltpu.SemaphoreType.DMA,
        pltpu.SemaphoreType.DMA,
    ],
)
def _sc_lookup(idx_hbm, tab_hbm, out_hbm, idxv, buf_a, buf_b,
               sg_a, sg_b, sw_a, sw_b):
    wid = lax.axis_index("s") * NC + lax.axis_index("c")
    base = wid * ROWS_PER_W

    pltpu.sync_copy(idx_hbm.at[pl.ds(base, ROWS_PER_W)], idxv)

    def remap_body(j, carry):
        v = idxv[pl.ds(j * L, L)]
        idxv[pl.ds(j * L, L)] = v + jnp.where(v < UPDATE_N, 0, UPDATE_N)
        return carry

    lax.fori_loop(0, ROWS_PER_W // L, remap_body, 0)

    bufs = (buf_a, buf_b)
    sgs = (sg_a, sg_b)
    sws = (sw_a, sw_b)

    def issue_gather(ci):
        b = ci % 2
        copies = []
        for s in range(SUB):
            sl = pl.ds(ci * CHUNK + s * (CHUNK // SUB), CHUNK // SUB)
            dl = pl.ds(s * (CHUNK // SUB), CHUNK // SUB)
            copies.append(pltpu.async_copy(
                tab_hbm.at[idxv.at[sl]], bufs[b].at[dl], sgs[b]))
        return copies

    gathers = [None, None]
    writes = [None, None]
    gathers[0] = issue_gather(0)

    for ci in range(N_CHUNKS):
        b = ci % 2
        for c in gathers[b]:
            c.wait()
        if ci + 1 < N_CHUNKS:
            nb = (ci + 1) % 2
            if writes[nb] is not None:
                writes[nb].wait()
            gathers[nb] = issue_gather(ci + 1)
        writes[b] = pltpu.async_copy(
            bufs[b], out_hbm.at[pl.ds(base + ci * CHUNK, CHUNK)], sws[b])

    for w in writes:
        if w is not None:
            w.wait()


UROWS_PER_W = UPDATE_N // NW
FROWS_PER_W = VOCAB_N // NW
CP_TILE = 625


@functools.partial(
    pl.kernel,
    out_type=jax.ShapeDtypeStruct((UPDATE_N + VOCAB_N, D), jnp.float32),
    mesh=_mesh,
    compiler_params=pltpu.CompilerParams(use_tc_tiling_on_sc=False),
    scratch_types=[
        pltpu.VMEM((CP_TILE, D), jnp.float32),
        pltpu.VMEM((CP_TILE, D), jnp.float32),
        pltpu.SemaphoreType.DMA,
        pltpu.SemaphoreType.DMA,
    ],
)
def _sc_concat(wu_hbm, wf_hbm, tab_hbm, tile_a, tile_b, sem_a, sem_b):
    wid = lax.axis_index("s") * NC + lax.axis_index("c")

    passes = []
    ubase = wid * UROWS_PER_W
    for t in range(UROWS_PER_W // CP_TILE):
        passes.append((wu_hbm, ubase + t * CP_TILE, ubase + t * CP_TILE))
    fbase = wid * FROWS_PER_W
    for t in range(FROWS_PER_W // CP_TILE):
        passes.append((wf_hbm, fbase + t * CP_TILE,
                       UPDATE_N + fbase + t * CP_TILE))

    tiles = (tile_a, tile_b)
    sems = (sem_a, sem_b)
    reads = [None, None]
    writes = [None, None]
    reads[0] = pltpu.async_copy(
        passes[0][0].at[pl.ds(passes[0][1], CP_TILE)], tile_a, sem_a)

    for i in range(len(passes)):
        b = i % 2
        reads[b].wait()
        if i + 1 < len(passes):
            nb = (i + 1) % 2
            if writes[nb] is not None:
                writes[nb].wait()
            src, s0, _ = passes[i + 1]
            reads[nb] = pltpu.async_copy(
                src.at[pl.ds(s0, CP_TILE)], tiles[nb], sems[nb])
        writes[b] = pltpu.async_copy(
            tiles[b], tab_hbm.at[pl.ds(passes[i][2], CP_TILE)], sems[b])

    for w in writes:
        if w is not None:
            w.wait()


def kernel(inp, W_update, W_fixed):
    idx = inp.reshape(B_ROWS).astype(jnp.int32)
    tab = _sc_concat(W_update, W_fixed)
    out = _sc_lookup(idx, tab)
    return out.reshape(inp.shape[0], inp.shape[1], D)


# speedup vs baseline: 7.9810x; 1.0531x over previous
import functools

import jax
import jax.numpy as jnp
from jax import lax
from jax.experimental import pallas as pl
from jax.experimental.pallas import tpu as pltpu
from jax.experimental.pallas import tpu_sc as plsc

UPDATE_N = 100000
VOCAB_N = 1000000
D = 32
L = 16
NC, NS = 2, 16
NW = NC * NS
B_ROWS = 4096 * 200
ROWS_PER_W = B_ROWS // NW
CHUNK = 1024
N_CHUNKS = ROWS_PER_W // CHUNK
SUB = 8

_mesh = plsc.VectorSubcoreMesh(core_axis_name="c", subcore_axis_name="s")


@functools.partial(
    pl.kernel,
    out_type=jax.ShapeDtypeStruct((B_ROWS, D), jnp.float32),
    mesh=_mesh,
    compiler_params=pltpu.CompilerParams(use_tc_tiling_on_sc=False),
    scratch_types=[
        pltpu.VMEM((ROWS_PER_W,), jnp.int32),
        pltpu.VMEM((CHUNK, D), jnp.float32),
        pltpu.VMEM((CHUNK, D), jnp.float32),
        pltpu.SemaphoreType.DMA,
        pltpu.SemaphoreType.DMA,
        pltpu.SemaphoreType.DMA,
        pltpu.SemaphoreType.DMA,
    ],
)
def _sc_lookup(idx_hbm, tab_hbm, out_hbm, idxv, buf_a, buf_b,
               sg_a, sg_b, sw_a, sw_b):
    wid = lax.axis_index("s") * NC + lax.axis_index("c")
    base = wid * ROWS_PER_W

    pltpu.sync_copy(idx_hbm.at[pl.ds(base, ROWS_PER_W)], idxv)

    def remap_body(j, carry):
        v = idxv[pl.ds(j * L, L)]
        idxv[pl.ds(j * L, L)] = v + jnp.where(v < UPDATE_N, 0, UPDATE_N)
        return carry

    lax.fori_loop(0, ROWS_PER_W // L, remap_body, 0)

    bufs = (buf_a, buf_b)
    sgs = (sg_a, sg_b)
    sws = (sw_a, sw_b)

    def issue_gather(ci):
        b = ci % 2
        copies = []
        for s in range(SUB):
            sl = pl.ds(ci * CHUNK + s * (CHUNK // SUB), CHUNK // SUB)
            dl = pl.ds(s * (CHUNK // SUB), CHUNK // SUB)
            copies.append(pltpu.async_copy(
                tab_hbm.at[idxv.at[sl]], bufs[b].at[dl], sgs[b]))
        return copies

    gathers = [None, None]
    writes = [None, None]
    gathers[0] = issue_gather(0)

    for ci in range(N_CHUNKS):
        b = ci % 2
        for c in gathers[b]:
            c.wait()
        if ci + 1 < N_CHUNKS:
            nb = (ci + 1) % 2
            if writes[nb] is not None:
                writes[nb].wait()
            gathers[nb] = issue_gather(ci + 1)
        writes[b] = pltpu.async_copy(
            bufs[b], out_hbm.at[pl.ds(base + ci * CHUNK, CHUNK)], sws[b])

    for w in writes:
        if w is not None:
            w.wait()


def kernel(inp, W_update, W_fixed):
    idx = inp.reshape(B_ROWS).astype(jnp.int32)
    tab = jnp.concatenate([W_update, W_fixed])
    out = _sc_lookup(idx, tab)
    return out.reshape(inp.shape[0], inp.shape[1], D)
